# Initial kernel scaffold; baseline (speedup 1.0000x reference)
import jax
import jax.numpy as jnp
from jax.experimental import pallas as pl


def kernel(x_esm, x_reschem, x_role, x_dist_raw, x_misc, x_vec, pos, edge_dist_raw, edge_index, params):
    raise NotImplementedError("write your pallas kernel here")


# speedup vs baseline: 26.8461x; 26.8461x over previous
import functools

import jax
import jax.numpy as jnp
from jax import lax
from jax.experimental import pallas as pl
from jax.experimental.pallas import tpu as pltpu
from jax.experimental.pallas import tpu_sc as plsc

N_NODES = 10000
N_EDGES = 640000
HID_S = 128
HID_V = 16
E_HID = 64
N_RBF = 16
GAMMA = 1.0 / ((12.0 / N_RBF) ** 2 + 1e-8)

W1L = 192
WG = 176

NSUB = 32
EPW = N_EDGES // NSUB
CHUNK = 200
NCHUNK = EPW // CHUNK


def _silu(x):
    return x * jax.nn.sigmoid(x)


def _ln(x, g, b):
    m = jnp.mean(x, axis=-1, keepdims=True)
    var = jnp.mean((x - m) ** 2, axis=-1, keepdims=True)
    return (x - m) / jnp.sqrt(var + 1e-5) * g + b


def _rbf16(col):
    centers = lax.broadcasted_iota(jnp.int32, (1, N_RBF), 1).astype(
        jnp.float32) * (12.0 / (N_RBF - 1))
    return jnp.exp(-GAMMA * (col - centers) ** 2)


def _vnorm48(v48):
    ss = v48[:, 0:16] ** 2 + v48[:, 16:32] ** 2 + v48[:, 32:48] ** 2
    return jnp.sqrt(jnp.clip(ss, 1e-8))


def _rep3(x):
    return jnp.concatenate([x, x, x], axis=-1)


def _node_encode_body(xesm, small, xdist, xvec3, wesm, besm, gesm, lbesm,
                      woesm, wosmall, worbf, bo, go, lbo, ivw, s_out, v_out):
    esm = _silu(_ln(jnp.dot(xesm[...], wesm[...],
                            preferred_element_type=jnp.float32) + besm[...],
                    gesm[...], lbesm[...]))
    xd = xdist[...]
    drbf = jnp.concatenate([_rbf16(xd[:, k:k + 1]) for k in range(4)], axis=-1)
    xo = (jnp.dot(esm, woesm[...], preferred_element_type=jnp.float32)
          + jnp.dot(small[...], wosmall[...], preferred_element_type=jnp.float32)
          + jnp.dot(drbf, worbf[...], preferred_element_type=jnp.float32)
          + bo[...])
    s_out[...] = _silu(_ln(xo, go[...], lbo[...]))
    xv = xvec3[...]
    v_out[...] = jnp.concatenate([xv[:, c:c + 1] * ivw[...] for c in range(3)],
                                 axis=-1)


def _node_encode(xesm, small, xdist, xvec3, w):
    nb = 5
    bn = N_NODES // nb
    full = lambda shape: pl.BlockSpec(shape, lambda i: (0, 0))
    row = lambda cdim: pl.BlockSpec((bn, cdim), lambda i: (i, 0))
    return pl.pallas_call(
        _node_encode_body,
        grid=(nb,),
        in_specs=[row(320), row(11), row(4), row(3),
                  full((320, 128)), full((1, 128)), full((1, 128)), full((1, 128)),
                  full((128, 128)), full((11, 128)), full((64, 128)),
                  full((1, 128)), full((1, 128)), full((1, 128)), full((1, 16))],
        out_specs=[row(128), row(48)],
        out_shape=[jax.ShapeDtypeStruct((N_NODES, 128), jnp.float32),
                   jax.ShapeDtypeStruct((N_NODES, 48), jnp.float32)],
    )(xesm, small, xdist, xvec3, *w)


def _prep_body(s, v48, pos, wsrc, wdst, b1, wvs, wvd,
               ta_src, tb_src, ta_dst, tb_dst):
    vn = _vnorm48(v48[...])
    xin = jnp.concatenate([s[...], vn], axis=-1)
    ta_src[...] = jnp.dot(xin, wsrc[...], preferred_element_type=jnp.float32)
    ta_dst[...] = jnp.dot(xin, wdst[...],
                          preferred_element_type=jnp.float32) + b1[...]
    v = v48[...]
    pvs = jnp.concatenate(
        [jnp.dot(v[:, 16 * c:16 * c + 16], wvs[...],
                 preferred_element_type=jnp.float32) for c in range(3)], axis=-1)
    pvd = jnp.concatenate(
        [jnp.dot(v[:, 16 * c:16 * c + 16], wvd[...],
                 preferred_element_type=jnp.float32) for c in range(3)], axis=-1)
    pad = jnp.zeros((s.shape[0], 128 - 51), jnp.float32)
    tb_src[...] = jnp.concatenate([pvs, pos[...], pad], axis=-1)
    tb_dst[...] = jnp.concatenate([pvd, pos[...], pad], axis=-1)


def _prep(s, v48, pos, w):
    nb = 5
    bn = N_NODES // nb
    full = lambda shape: pl.BlockSpec(shape, lambda i: (0, 0))
    row = lambda cdim: pl.BlockSpec((bn, cdim), lambda i: (i, 0))
    return pl.pallas_call(
        _prep_body,
        grid=(nb,),
        in_specs=[row(128), row(48), row(3),
                  full((144, 128)), full((144, 128)), full((1, 128)),
                  full((16, 16)), full((16, 16))],
        out_specs=[row(128), row(128), row(128), row(128)],
        out_shape=[jax.ShapeDtypeStruct((N_NODES, 128), jnp.float32)] * 4,
    )(s, v48, pos, *w)


def _edge_core(g, edge_s, el, reln, wse, wlen, w2, b2, wg, bg, wve):
    pre = (g[0] + g[1]
           + jnp.dot(edge_s, wse[...], preferred_element_type=jnp.float32)
           + el * wlen[...])
    h = _silu(pre)
    m_s = jnp.dot(h, w2[...], preferred_element_type=jnp.float32) + b2[...]
    gate = jax.nn.sigmoid(
        jnp.dot(m_s, wg[...], preferred_element_type=jnp.float32) + bg[...])
    ev = jnp.concatenate([reln[:, c:c + 1] * wve[...] for c in range(3)], axis=-1)
    m_v = (g[2][:, 0:48] + g[3][:, 0:48] + ev) * _rep3(gate)
    pad = jnp.zeros((m_v.shape[0], 80), jnp.float32)
    return jnp.concatenate([m_s[None], jnp.concatenate([m_v, pad], -1)[None]], 0)


def _edge_l1_body(g2, edr, wer, ber, ger, lber, wse, wlen, w2, b2, wg, bg,
                  wve, m_out, ef_out):
    g = g2[...]
    ed = edr[...]
    erbf = jnp.concatenate([_rbf16(ed[:, k:k + 1]) for k in range(2)], axis=-1)
    edge_s = _silu(_ln(jnp.dot(erbf, wer[...],
                               preferred_element_type=jnp.float32) + ber[...],
                       ger[...], lber[...]))
    rel = g[3][:, 48:51] - g[2][:, 48:51]
    rn2 = jnp.sum(rel * rel, axis=-1, keepdims=True)
    reln = rel / jnp.sqrt(jnp.clip(rn2, 1e-8))
    el = jnp.sqrt(jnp.clip(jnp.sum(reln * reln, axis=-1, keepdims=True), 1e-8))
    m_out[...] = _edge_core(g, edge_s, el, reln, wse, wlen, w2, b2, wg, bg, wve)
    ef_out[...] = jnp.concatenate([edge_s, el, reln], axis=-1)


def _edge_l1(g2, edr, w):
    nb = 200
    be = N_EDGES // nb
    full = lambda shape: pl.BlockSpec(shape, lambda i: (0, 0))
    return pl.pallas_call(
        _edge_l1_body,
        grid=(nb,),
        in_specs=[pl.BlockSpec((4, be, 128), lambda i: (0, i, 0)),
                  pl.BlockSpec((be, 2), lambda i: (i, 0)),
                  full((32, 64)), full((1, 64)), full((1, 64)), full((1, 64)),
                  full((64, 128)), full((1, 128)),
                  full((128, 128)), full((1, 128)),
                  full((128, 16)), full((1, 16)), full((1, 16))],
        out_specs=[pl.BlockSpec((2, be, 128), lambda i: (0, i, 0)),
                   pl.BlockSpec((be, 68), lambda i: (i, 0))],
        out_shape=[jax.ShapeDtypeStruct((2, N_EDGES, 128), jnp.float32),
                   jax.ShapeDtypeStruct((N_EDGES, 68), jnp.float32)],
    )(g2, edr, *w)


def _edge_body(g2, ef, wse, wlen, w2, b2, wg, bg, wve, m_out):
    g = g2[...]
    e = ef[...]
    edge_s = e[:, 0:64]
    el = e[:, 64:65]
    reln = e[:, 65:68]
    m_out[...] = _edge_core(g, edge_s, el, reln, wse, wlen, w2, b2, wg, bg, wve)


def _edge(g2, ef, w):
    nb = 200
    be = N_EDGES // nb
    full = lambda shape: pl.BlockSpec(shape, lambda i: (0, 0))
    return pl.pallas_call(
        _edge_body,
        grid=(nb,),
        in_specs=[pl.BlockSpec((4, be, 128), lambda i: (0, i, 0)),
                  pl.BlockSpec((be, 68), lambda i: (i, 0)),
                  full((64, 128)), full((1, 128)),
                  full((128, 128)), full((1, 128)),
                  full((128, 16)), full((1, 16)), full((1, 16))],
        out_specs=pl.BlockSpec((2, be, 128), lambda i: (0, i, 0)),
        out_shape=jax.ShapeDtypeStruct((2, N_EDGES, 128), jnp.float32),
    )(g2, ef, *w)


def _upd_body(s, v48, agga, aggb, w1u, b1u, w2u, b2u, wgu, bgu, wvv, wva,
              lng, lnb, s_out, v_out):
    a0 = agga[...]
    b0 = aggb[...]
    aggs = a0[0] + a0[1]
    aggv = b0[0][:, 0:48] + b0[1][:, 0:48]
    v = v48[...]
    vn_v = _vnorm48(v)
    vn_a = _vnorm48(aggv)
    xin = jnp.concatenate([s[...], aggs, vn_v, vn_a], axis=-1)
    pre = jnp.dot(xin, w1u[...], preferred_element_type=jnp.float32) + b1u[...]
    h = _silu(pre)
    ds_ = jnp.dot(h, w2u[...], preferred_element_type=jnp.float32) + b2u[...]
    gu = jax.nn.sigmoid(
        jnp.dot(ds_, wgu[...], preferred_element_type=jnp.float32) + bgu[...])
    dv = jnp.concatenate(
        [jnp.dot(v[:, 16 * c:16 * c + 16], wvv[...],
                 preferred_element_type=jnp.float32)
         + jnp.dot(aggv[:, 16 * c:16 * c + 16], wva[...],
                   preferred_element_type=jnp.float32) for c in range(3)],
        axis=-1)
    s_out[...] = _ln(s[...] + ds_, lng[...], lnb[...])
    v_out[...] = v + dv * _rep3(gu)


def _upd(s, v48, agga, aggb, w):
    nb = 5
    bn = N_NODES // nb
    full = lambda shape: pl.BlockSpec(shape, lambda i: (0, 0))
    row = lambda cdim: pl.BlockSpec((bn, cdim), lambda i: (i, 0))
    agg_spec = pl.BlockSpec((2, bn, 128), lambda i: (0, i, 0))
    return pl.pallas_call(
        _upd_body,
        grid=(nb,),
        in_specs=[row(128), row(48), agg_spec, agg_spec,
                  full((288, 128)), full((1, 128)),
                  full((128, 128)), full((1, 128)),
                  full((128, 16)), full((1, 16)),
                  full((16, 16)), full((16, 16)),
                  full((1, 128)), full((1, 128))],
        out_specs=[row(128), row(48)],
        out_shape=[jax.ShapeDtypeStruct((N_NODES, 128), jnp.float32),
                   jax.ShapeDtypeStruct((N_NODES, 48), jnp.float32)],
    )(s, v48, agga, aggb, *w)


def _heads_body(s, w1m, b1m, w2m, b2m, w1e, b1e, w2e, b2e, metal, ec):
    sv = s[...]
    g = jnp.concatenate([jnp.mean(sv, axis=0, keepdims=True),
                         jnp.max(sv, axis=0, keepdims=True)], axis=-1)
    hm = _silu(jnp.dot(g, w1m[...], preferred_element_type=jnp.float32) + b1m[...])
    metal[...] = jnp.dot(hm, w2m[...], preferred_element_type=jnp.float32) + b2m[...]
    he = _silu(jnp.dot(g, w1e[...], preferred_element_type=jnp.float32) + b1e[...])
    ec[...] = jnp.dot(he, w2e[...], preferred_element_type=jnp.float32) + b2e[...]


def _heads(s, w):
    full = lambda shape: pl.BlockSpec(shape, lambda i: (0, 0))
    return pl.pallas_call(
        _heads_body,
        grid=(1,),
        in_specs=[pl.BlockSpec((N_NODES, 128), lambda i: (0, 0)),
                  full((256, 128)), full((1, 128)), full((128, 8)), full((1, 8)),
                  full((256, 128)), full((1, 128)), full((128, 7)), full((1, 7))],
        out_specs=[full((1, 8)), full((1, 7))],
        out_shape=[jax.ShapeDtypeStruct((1, 8), jnp.float32),
                   jax.ShapeDtypeStruct((1, 7), jnp.float32)],
    )(s, *w)


def _sc_gather(ta_src, tb_src, ta_dst, tb_dst, src, dst):
    mesh = plsc.VectorSubcoreMesh(core_axis_name="c", subcore_axis_name="s")

    @functools.partial(
        pl.kernel,
        out_type=jax.ShapeDtypeStruct((4, N_EDGES, 128), jnp.float32),
        mesh=mesh,
        scratch_types=[
            pltpu.VMEM((CHUNK,), jnp.int32),
            pltpu.VMEM((CHUNK,), jnp.int32),
            pltpu.VMEM((CHUNK, 128), jnp.float32),
            pltpu.VMEM((CHUNK, 128), jnp.float32),
            pltpu.VMEM((CHUNK, 128), jnp.float32),
            pltpu.VMEM((CHUNK, 128), jnp.float32),
            pltpu.SemaphoreType.DMA,
            pltpu.SemaphoreType.DMA,
            pltpu.SemaphoreType.DMA,
            pltpu.SemaphoreType.DMA,
        ],
    )
    def k(tas, tbs, tad, tbd, si, di, out, isv, idv, b0, b1, b2, b3,
          s0, s1, s2, s3):
        w = lax.axis_index("s") * 2 + lax.axis_index("c")
        base0 = w * EPW

        def body(i, carry):
            base = base0 + i * CHUNK
            pltpu.sync_copy(si.at[pl.ds(base, CHUNK)], isv)
            pltpu.sync_copy(di.at[pl.ds(base, CHUNK)], idv)
            c0 = pltpu.async_copy(tas.at[isv], b0, s0)
            c1 = pltpu.async_copy(tad.at[idv], b1, s1)
            c2 = pltpu.async_copy(tbs.at[isv], b2, s2)
            c3 = pltpu.async_copy(tbd.at[idv], b3, s3)
            c0.wait()
            pltpu.sync_copy(b0, out.at[0, pl.ds(base, CHUNK)])
            c1.wait()
            pltpu.sync_copy(b1, out.at[1, pl.ds(base, CHUNK)])
            c2.wait()
            pltpu.sync_copy(b2, out.at[2, pl.ds(base, CHUNK)])
            c3.wait()
            pltpu.sync_copy(b3, out.at[3, pl.ds(base, CHUNK)])
            return carry

        lax.fori_loop(0, NCHUNK, body, 0)

    return k(ta_src, tb_src, ta_dst, tb_dst, src, dst)


def _sc_scatter(m, dst):
    mesh = plsc.VectorSubcoreMesh(core_axis_name="c", subcore_axis_name="s")
    epc = N_EDGES // 2
    epw = epc // 16
    nchunk = epw // CHUNK
    rb = 80

    @functools.partial(
        pl.kernel,
        out_type=jax.ShapeDtypeStruct((2, N_NODES, 128), jnp.float32),
        mesh=mesh,
        scratch_types=[
            pltpu.VMEM((CHUNK,), jnp.int32),
            pltpu.VMEM((CHUNK, 128), jnp.float32),
            pltpu.VMEM((rb, 128), jnp.float32),
            pltpu.VMEM_SHARED((N_NODES, 128), jnp.float32),
        ],
    )
    def k(mm, di, out, idxv, mbuf, stage, acc):
        c = lax.axis_index("c")
        s = lax.axis_index("s")

        row0 = s * 640
        nrow_chunks = lax.select(s < 15, 8, 5)

        def zero_col(j, _):
            for kk in range(8):
                stage[j, pl.ds(kk * 16, 16)] = jnp.zeros((16,), jnp.float32)
            return _

        lax.fori_loop(0, rb, zero_col, 0)

        def zrow(t, _):
            pltpu.sync_copy(stage, acc.at[pl.ds(row0 + t * rb, rb)])
            return _

        lax.fori_loop(0, nrow_chunks, zrow, 0)
        plsc.subcore_barrier()

        base0 = c * epc + s * epw

        def body(i, carry):
            base = base0 + i * CHUNK
            pltpu.sync_copy(di.at[pl.ds(base, CHUNK)], idxv)
            pltpu.sync_copy(mm.at[pl.ds(base, CHUNK)], mbuf)
            pltpu.sync_copy(mbuf, acc.at[idxv], add=True)
            return carry

        lax.fori_loop(0, nchunk, body, 0)
        plsc.subcore_barrier()

        def wrow(t, _):
            r = row0 + t * rb
            pltpu.sync_copy(acc.at[pl.ds(r, rb)], stage)
            pltpu.sync_copy(stage, out.at[c, pl.ds(r, rb)])
            return _

        lax.fori_loop(0, nrow_chunks, wrow, 0)

    return k(m, dst)


def _prep_weights(params):
    pn = params["node_enc"]
    enc = [
        pn["esm_lin"]["W"].T, pn["esm_lin"]["b"][None, :],
        pn["esm_ln"]["g"][None, :], pn["esm_ln"]["b"][None, :],
        pn["out_lin"]["W"][:, 0:128].T, pn["out_lin"]["W"][:, 128:139].T,
        pn["out_lin"]["W"][:, 139:203].T, pn["out_lin"]["b"][None, :],
        pn["out_ln"]["g"][None, :], pn["out_ln"]["b"][None, :],
        params["init_vec_W"][:, 0][None, :],
    ]
    layers = []
    for lp in params["layers"]:
        mp = lp["msg"]
        W1 = mp["mlp1"]["W"]
        Wv = mp["Wv"]
        prep_w = [
            jnp.concatenate([W1[:, 0:128], W1[:, 321:337]], axis=1).T,
            jnp.concatenate([W1[:, 128:256], W1[:, 337:353]], axis=1).T,
            mp["mlp1"]["b"][None, :],
            Wv[:, 0:16].T, Wv[:, 16:32].T,
        ]
        edge_w = [
            W1[:, 256:320].T, (W1[:, 320] + W1[:, 353])[None, :],
            mp["mlp2"]["W"].T, mp["mlp2"]["b"][None, :],
            mp["gate"]["W"].T, mp["gate"]["b"][None, :],
            Wv[:, 32][None, :],
        ]
        up = lp["upd"]
        upd_w = [
            up["mlp1"]["W"].T, up["mlp1"]["b"][None, :],
            up["mlp2"]["W"].T, up["mlp2"]["b"][None, :],
            up["gate"]["W"].T, up["gate"]["b"][None, :],
            up["Wv"][:, 0:16].T, up["Wv"][:, 16:32].T,
            lp["ln"]["g"][None, :], lp["ln"]["b"][None, :],
        ]
        layers.append((prep_w, edge_w, upd_w))
    pe = params["edge_enc"]
    enc_e = [
        pe["lin"]["W"].T, pe["lin"]["b"][None, :],
        pe["ln"]["g"][None, :], pe["ln"]["b"][None, :],
    ]
    hm = params["head_metal"]
    he = params["head_ec"]
    heads = [
        hm["l1"]["W"].T, hm["l1"]["b"][None, :],
        hm["l2"]["W"].T, hm["l2"]["b"][None, :],
        he["l1"]["W"].T, he["l1"]["b"][None, :],
        he["l2"]["W"].T, he["l2"]["b"][None, :],
    ]
    return enc, enc_e, layers, heads


def kernel(x_esm, x_reschem, x_role, x_dist_raw, x_misc, x_vec, pos,
           edge_dist_raw, edge_index, params):
    enc, enc_e, layers, heads = _prep_weights(params)
    small = jnp.concatenate([x_reschem, x_role, x_misc], axis=-1)
    xvec3 = x_vec[:, 0, :]
    src = edge_index[0]
    dst = edge_index[1]

    s, v48 = _node_encode(x_esm, small, x_dist_raw, xvec3, enc)

    ef = None
    for li, (prep_w, edge_w, upd_w) in enumerate(layers):
        first = li == 0
        ta_src, tb_src, ta_dst, tb_dst = _prep(s, v48, pos, prep_w)
        g2 = _sc_gather(ta_src, tb_src, ta_dst, tb_dst, src, dst)
        if first:
            m, ef = _edge_l1(g2, edge_dist_raw, enc_e + edge_w)
        else:
            m = _edge(g2, ef, edge_w)
        agga = _sc_scatter(m[0], dst)
        aggb = _sc_scatter(m[1], dst)
        s, v48 = _upd(s, v48, agga, aggb, upd_w)

    metal, ec = _heads(s, heads)
    return (metal[0], ec[0])
